# baseline (speedup 1.0000x reference)
import functools

import jax
import jax.numpy as jnp
from jax import lax
from jax.experimental import pallas as pl
from jax.experimental.pallas import tpu as pltpu
from jax.experimental.pallas import tpu_sc as plsc

NUM_ROWS = 1000000
EMB_DIM = 32
BATCH = 16384

_NC, _NS, _L = 2, 16, 16
_NW = _NC * _NS
_CHUNK = BATCH // _NW
_GROUPS = _CHUNK // _L
_FLAT = EMB_DIM * _CHUNK


def _rsqrt(x):
    i = plsc.bitcast(x, jnp.int32)
    i = 0x5F3759DF - (i >> 1)
    y = plsc.bitcast(i, jnp.float32)
    for _ in range(3):
        y = y * (1.5 - 0.5 * x * y * y)
    return y


def _scale(ns):
    norm = ns * _rsqrt(ns)
    return jnp.where(ns > 1.0, 1.0 / (norm + 1e-7), jnp.float32(1.0))


def _fill_flat_indices(idx_v, fidx_v):
    def fill(j, carry):
        v = idx_v[pl.ds(j * _L, _L)]
        for d in range(EMB_DIM):
            fidx_v[pl.ds(d * _CHUNK + j * _L, _L)] = v + d * NUM_ROWS
        return carry

    lax.fori_loop(0, _GROUPS, fill, 0, unroll=False)


def _bpr_body(uidx_hbm, pidx_hbm, nidx_hbm, ut_hbm, it_hbm, out_hbm,
              uidx_v, fidx_u, fidx_p, fidx_n,
              u_rows, p_rows, n_rows, out_v, sem):
    wid = lax.axis_index("s") * _NC + lax.axis_index("c")
    base = wid * _CHUNK

    pltpu.sync_copy(uidx_hbm.at[pl.ds(base, _CHUNK)], uidx_v)
    _fill_flat_indices(uidx_v, fidx_u)
    cu = pltpu.async_copy(ut_hbm.at[fidx_u], u_rows, sem)
    pltpu.sync_copy(pidx_hbm.at[pl.ds(base, _CHUNK)], uidx_v)
    _fill_flat_indices(uidx_v, fidx_p)
    cp = pltpu.async_copy(it_hbm.at[fidx_p], p_rows, sem)
    pltpu.sync_copy(nidx_hbm.at[pl.ds(base, _CHUNK)], uidx_v)
    _fill_flat_indices(uidx_v, fidx_n)
    cn = pltpu.async_copy(it_hbm.at[fidx_n], n_rows, sem)
    cu.wait()
    cp.wait()
    cn.wait()

    def group(g, carry):
        sl = pl.ds(g * _L, _L)
        zero = jnp.zeros((_L,), jnp.float32)
        ns_u = ns_p = ns_n = dp = dn = zero
        for d in range(EMB_DIM):
            u = u_rows[pl.ds(d * _CHUNK + g * _L, _L)]
            p = p_rows[pl.ds(d * _CHUNK + g * _L, _L)]
            n = n_rows[pl.ds(d * _CHUNK + g * _L, _L)]
            ns_u = ns_u + u * u
            ns_p = ns_p + p * p
            ns_n = ns_n + n * n
            dp = dp + u * p
            dn = dn + u * n
        su = _scale(ns_u)
        sp = _scale(ns_p)
        sn = _scale(ns_n)
        x = su * (sp * dp - sn * dn)
        out_v[sl] = 1.0 / (1.0 + jnp.exp(-x))
        return carry

    lax.fori_loop(0, _GROUPS, group, 0, unroll=False)
    pltpu.sync_copy(out_v, out_hbm.at[pl.ds(base, _CHUNK)])


def kernel(positive, negative, user_track_count, user_table, item_table):
    del user_track_count
    uidx = positive[:, 0].astype(jnp.int32)
    pidx = positive[:, 1].astype(jnp.int32)
    nidx = negative[:, 1].astype(jnp.int32)
    ut = user_table.T.reshape(-1)
    it = item_table.T.reshape(-1)

    mesh = plsc.VectorSubcoreMesh(core_axis_name="c", subcore_axis_name="s")
    run = pl.kernel(
        _bpr_body,
        out_type=jax.ShapeDtypeStruct((BATCH,), jnp.float32),
        mesh=mesh,
        compiler_params=pltpu.CompilerParams(
            needs_layout_passes=False, use_tc_tiling_on_sc=False),
        scratch_types=[
            pltpu.VMEM((_CHUNK,), jnp.int32),
            pltpu.VMEM((_FLAT,), jnp.int32),
            pltpu.VMEM((_FLAT,), jnp.int32),
            pltpu.VMEM((_FLAT,), jnp.int32),
            pltpu.VMEM((_FLAT,), jnp.float32),
            pltpu.VMEM((_FLAT,), jnp.float32),
            pltpu.VMEM((_FLAT,), jnp.float32),
            pltpu.VMEM((_CHUNK,), jnp.float32),
            pltpu.SemaphoreType.DMA,
        ],
    )
    return run(uidx, pidx, nidx, ut, it)


# speedup vs baseline: 1.0007x; 1.0007x over previous
import functools

import jax
import jax.numpy as jnp
from jax import lax
from jax.experimental import pallas as pl
from jax.experimental.pallas import tpu as pltpu
from jax.experimental.pallas import tpu_sc as plsc

NUM_ROWS = 1000000
EMB_DIM = 32
BATCH = 16384

_NC, _NS, _L = 2, 16, 16
_NW = _NC * _NS
_CHUNK = BATCH // _NW
_GROUPS = _CHUNK // _L
_LAG = 3


def _rsqrt(x):
    i = plsc.bitcast(x, jnp.int32)
    i = 0x5F3759DF - (i >> 1)
    y = plsc.bitcast(i, jnp.float32)
    for _ in range(3):
        y = y * (1.5 - 0.5 * x * y * y)
    return y


def _scale(ns):
    norm = ns * _rsqrt(ns)
    return jnp.where(ns > 1.0, 1.0 / (norm + 1e-7), jnp.float32(1.0))


def _bpr_body(uidx_hbm, pidx_hbm, nidx_hbm, ut_hbm, it_hbm, out_hbm,
              uidx_v, pidx_v, nidx_v, u_rows, p_rows, n_rows, out_v, sem):
    wid = lax.axis_index("s") * _NC + lax.axis_index("c")
    base = wid * _CHUNK

    pltpu.sync_copy(uidx_hbm.at[pl.ds(base, _CHUNK)], uidx_v)
    pltpu.sync_copy(pidx_hbm.at[pl.ds(base, _CHUNK)], pidx_v)
    pltpu.sync_copy(nidx_hbm.at[pl.ds(base, _CHUNK)], nidx_v)

    def enqueue(g):
        sl = pl.ds(g * _L, _L)
        uv = uidx_v[sl]
        pv = pidx_v[sl]
        nv = nidx_v[sl]
        for d in range(EMB_DIM):
            dst = pl.ds(d * _CHUNK + g * _L, _L)
            off = d * NUM_ROWS
            pltpu.async_copy(ut_hbm.at[uv + off], u_rows.at[dst], sem)
            pltpu.async_copy(it_hbm.at[pv + off], p_rows.at[dst], sem)
            pltpu.async_copy(it_hbm.at[nv + off], n_rows.at[dst], sem)

    def drain(g):
        src = ut_hbm.at[pl.ds(0, _L)]
        for d in range(EMB_DIM):
            dst = pl.ds(d * _CHUNK + g * _L, _L)
            pltpu.make_async_copy(src, u_rows.at[dst], sem).wait()
            pltpu.make_async_copy(src, p_rows.at[dst], sem).wait()
            pltpu.make_async_copy(src, n_rows.at[dst], sem).wait()

    def gather_step(g, carry):
        @pl.when(g < _GROUPS)
        def _():
            enqueue(g)

        @pl.when(g >= _LAG)
        def _():
            drain(g - _LAG)

        return carry

    lax.fori_loop(0, _GROUPS + _LAG, gather_step, 0, unroll=False)

    def group(g, carry):
        sl = pl.ds(g * _L, _L)
        zero = jnp.zeros((_L,), jnp.float32)
        ns_u = ns_p = ns_n = dp = dn = zero
        for d in range(EMB_DIM):
            u = u_rows[pl.ds(d * _CHUNK + g * _L, _L)]
            p = p_rows[pl.ds(d * _CHUNK + g * _L, _L)]
            n = n_rows[pl.ds(d * _CHUNK + g * _L, _L)]
            ns_u = ns_u + u * u
            ns_p = ns_p + p * p
            ns_n = ns_n + n * n
            dp = dp + u * p
            dn = dn + u * n
        su = _scale(ns_u)
        sp = _scale(ns_p)
        sn = _scale(ns_n)
        x = su * (sp * dp - sn * dn)
        out_v[sl] = 1.0 / (1.0 + jnp.exp(-x))
        return carry

    lax.fori_loop(0, _GROUPS, group, 0, unroll=False)
    pltpu.sync_copy(out_v, out_hbm.at[pl.ds(base, _CHUNK)])


def kernel(positive, negative, user_track_count, user_table, item_table):
    del user_track_count
    uidx = positive[:, 0].astype(jnp.int32)
    pidx = positive[:, 1].astype(jnp.int32)
    nidx = negative[:, 1].astype(jnp.int32)
    ut = user_table.T.reshape(-1)
    it = item_table.T.reshape(-1)

    mesh = plsc.VectorSubcoreMesh(core_axis_name="c", subcore_axis_name="s")
    run = pl.kernel(
        _bpr_body,
        out_type=jax.ShapeDtypeStruct((BATCH,), jnp.float32),
        mesh=mesh,
        compiler_params=pltpu.CompilerParams(
            needs_layout_passes=False, use_tc_tiling_on_sc=False),
        scratch_types=[
            pltpu.VMEM((_CHUNK,), jnp.int32),
            pltpu.VMEM((_CHUNK,), jnp.int32),
            pltpu.VMEM((_CHUNK,), jnp.int32),
            pltpu.VMEM((EMB_DIM * _CHUNK,), jnp.float32),
            pltpu.VMEM((EMB_DIM * _CHUNK,), jnp.float32),
            pltpu.VMEM((EMB_DIM * _CHUNK,), jnp.float32),
            pltpu.VMEM((_CHUNK,), jnp.float32),
            pltpu.SemaphoreType.DMA,
        ],
    )
    return run(uidx, pidx, nidx, ut, it)


# speedup vs baseline: 4.7479x; 4.7448x over previous
import functools

import jax
import jax.numpy as jnp
from jax import lax
from jax.experimental import pallas as pl
from jax.experimental.pallas import tpu as pltpu
from jax.experimental.pallas import tpu_sc as plsc

NUM_ROWS = 1000000
EMB_DIM = 32
BATCH = 16384

_NC, _NS, _L = 2, 16, 16
_NW = _NC * _NS
_CHUNK = BATCH // _NW
_GROUPS = _CHUNK // _L


def _rsqrt(x):
    i = plsc.bitcast(x, jnp.int32)
    i = 0x5F3759DF - (i >> 1)
    y = plsc.bitcast(i, jnp.float32)
    for _ in range(3):
        y = y * (1.5 - 0.5 * x * y * y)
    return y


def _scale(ns):
    norm = ns * _rsqrt(ns)
    return jnp.where(ns > 1.0, 1.0 / (norm + 1e-7), jnp.float32(1.0))


def _bpr_body(uidx_hbm, pidx_hbm, nidx_hbm, ut_hbm, it_hbm, out_hbm,
              uidx_v, pidx_v, nidx_v, ub, pb, nb, uT, pT, nT, out_v, sem):
    wid = lax.axis_index("s") * _NC + lax.axis_index("c")
    base = wid * _CHUNK

    pltpu.sync_copy(uidx_hbm.at[pl.ds(base, _CHUNK)], uidx_v)
    pltpu.sync_copy(pidx_hbm.at[pl.ds(base, _CHUNK)], pidx_v)
    pltpu.sync_copy(nidx_hbm.at[pl.ds(base, _CHUNK)], nidx_v)

    cu = pltpu.async_copy(ut_hbm.at[uidx_v], ub, sem)
    cp = pltpu.async_copy(it_hbm.at[pidx_v], pb, sem)
    cn = pltpu.async_copy(it_hbm.at[nidx_v], nb, sem)
    cu.wait()
    cp.wait()
    cn.wait()

    lane = lax.iota(jnp.int32, _L)
    lane_hi = lane + _L

    def transpose_row(r, carry):
        col = jnp.full((_L,), r, jnp.int32)
        for src, dst in ((ub, uT), (pb, pT), (nb, nT)):
            v = src[r, :]
            a, b = plsc.unpack(v, format=plsc.PackFormat.INTERLEAVED)
            plsc.store_scatter(dst, [lane, col], a)
            plsc.store_scatter(dst, [lane_hi, col], b)
        return carry

    lax.fori_loop(0, _CHUNK, transpose_row, 0, unroll=False)

    def group(g, carry):
        sl = pl.ds(g * _L, _L)
        zero = jnp.zeros((_L,), jnp.float32)
        ns_u = ns_p = ns_n = dp = dn = zero
        for d in range(EMB_DIM):
            u = uT[d, sl]
            p = pT[d, sl]
            n = nT[d, sl]
            ns_u = ns_u + u * u
            ns_p = ns_p + p * p
            ns_n = ns_n + n * n
            dp = dp + u * p
            dn = dn + u * n
        su = _scale(ns_u)
        sp = _scale(ns_p)
        sn = _scale(ns_n)
        x = su * (sp * dp - sn * dn)
        out_v[sl] = 1.0 / (1.0 + jnp.exp(-x))
        return carry

    lax.fori_loop(0, _GROUPS, group, 0, unroll=False)
    pltpu.sync_copy(out_v, out_hbm.at[pl.ds(base, _CHUNK)])


def kernel(positive, negative, user_track_count, user_table, item_table):
    del user_track_count
    uidx = positive[:, 0].astype(jnp.int32)
    pidx = positive[:, 1].astype(jnp.int32)
    nidx = negative[:, 1].astype(jnp.int32)
    ut = user_table.astype(jnp.bfloat16)
    it = item_table.astype(jnp.bfloat16)

    mesh = plsc.VectorSubcoreMesh(core_axis_name="c", subcore_axis_name="s")
    run = pl.kernel(
        _bpr_body,
        out_type=jax.ShapeDtypeStruct((BATCH,), jnp.float32),
        mesh=mesh,
        compiler_params=pltpu.CompilerParams(
            needs_layout_passes=False, use_tc_tiling_on_sc=False),
        scratch_types=[
            pltpu.VMEM((_CHUNK,), jnp.int32),
            pltpu.VMEM((_CHUNK,), jnp.int32),
            pltpu.VMEM((_CHUNK,), jnp.int32),
            pltpu.VMEM((_CHUNK, EMB_DIM), jnp.bfloat16),
            pltpu.VMEM((_CHUNK, EMB_DIM), jnp.bfloat16),
            pltpu.VMEM((_CHUNK, EMB_DIM), jnp.bfloat16),
            pltpu.VMEM((EMB_DIM, _CHUNK), jnp.float32),
            pltpu.VMEM((EMB_DIM, _CHUNK), jnp.float32),
            pltpu.VMEM((EMB_DIM, _CHUNK), jnp.float32),
            pltpu.VMEM((_CHUNK,), jnp.float32),
            pltpu.SemaphoreType.DMA,
        ],
    )
    return run(uidx, pidx, nidx, ut, it)


# speedup vs baseline: 5.6176x; 1.1832x over previous
import functools

import jax
import jax.numpy as jnp
from jax import lax
from jax.experimental import pallas as pl
from jax.experimental.pallas import tpu as pltpu
from jax.experimental.pallas import tpu_sc as plsc

NUM_USER = 1000000
NUM_ITEM = 1000000
EMB_DIM = 32
BATCH = 16384

_NC, _NS, _L = 2, 16, 16
_NW = _NC * _NS
_CHUNK = BATCH // _NW
_GROUPS = _CHUNK // _L


def _rsqrt(x):
    i = plsc.bitcast(x, jnp.int32)
    i = 0x5F3759DF - (i >> 1)
    y = plsc.bitcast(i, jnp.float32)
    for _ in range(3):
        y = y * (1.5 - 0.5 * x * y * y)
    return y


def _scale(ns):
    norm = ns * _rsqrt(ns)
    return jnp.where(ns > 1.0, 1.0 / (norm + 1e-7), jnp.float32(1.0))


def _bpr_body(uidx_hbm, pidx_hbm, nidx_hbm, user_hbm, item_hbm, out_hbm,
              uidx_v, pidx_v, nidx_v, u_rows, p_rows, n_rows, out_v, sem):
    wid = lax.axis_index("s") * _NC + lax.axis_index("c")
    base = wid * _CHUNK

    pltpu.sync_copy(uidx_hbm.at[pl.ds(base, _CHUNK)], uidx_v)
    pltpu.sync_copy(pidx_hbm.at[pl.ds(base, _CHUNK)], pidx_v)
    pltpu.sync_copy(nidx_hbm.at[pl.ds(base, _CHUNK)], nidx_v)

    cu = pltpu.async_copy(user_hbm.at[uidx_v], u_rows, sem)
    cp = pltpu.async_copy(item_hbm.at[pidx_v], p_rows, sem)
    cn = pltpu.async_copy(item_hbm.at[nidx_v], n_rows, sem)
    cu.wait()
    cp.wait()
    cn.wait()

    lane = lax.iota(jnp.int32, _L)

    def group(g, carry):
        rows = jnp.full((_L,), g * _L, jnp.int32) + lane
        zero = jnp.zeros((_L,), jnp.float32)
        ns_u = ns_p = ns_n = dp = dn = zero
        for d in range(EMB_DIM):
            col = jnp.full((_L,), d, jnp.int32)
            u = plsc.load_gather(u_rows, [rows, col])
            p = plsc.load_gather(p_rows, [rows, col])
            n = plsc.load_gather(n_rows, [rows, col])
            ns_u = ns_u + u * u
            ns_p = ns_p + p * p
            ns_n = ns_n + n * n
            dp = dp + u * p
            dn = dn + u * n
        su = _scale(ns_u)
        sp = _scale(ns_p)
        sn = _scale(ns_n)
        x = su * (sp * dp - sn * dn)
        out_v[pl.ds(g * _L, _L)] = 1.0 / (1.0 + jnp.exp(-x))
        return carry

    lax.fori_loop(0, _GROUPS, group, 0, unroll=False)
    pltpu.sync_copy(out_v, out_hbm.at[pl.ds(base, _CHUNK)])


def kernel(positive, negative, user_track_count, user_table, item_table):
    del user_track_count
    uidx = positive[:, 0].astype(jnp.int32)
    pidx = positive[:, 1].astype(jnp.int32)
    nidx = negative[:, 1].astype(jnp.int32)

    mesh = plsc.VectorSubcoreMesh(core_axis_name="c", subcore_axis_name="s")
    run = pl.kernel(
        _bpr_body,
        out_type=jax.ShapeDtypeStruct((BATCH,), jnp.float32),
        mesh=mesh,
        compiler_params=pltpu.CompilerParams(
            needs_layout_passes=False, use_tc_tiling_on_sc=False),
        scratch_types=[
            pltpu.VMEM((_CHUNK,), jnp.int32),
            pltpu.VMEM((_CHUNK,), jnp.int32),
            pltpu.VMEM((_CHUNK,), jnp.int32),
            pltpu.VMEM((_CHUNK, EMB_DIM), jnp.float32),
            pltpu.VMEM((_CHUNK, EMB_DIM), jnp.float32),
            pltpu.VMEM((_CHUNK, EMB_DIM), jnp.float32),
            pltpu.VMEM((_CHUNK,), jnp.float32),
            pltpu.SemaphoreType.DMA,
        ],
    )
    return run(uidx, pidx, nidx, user_table, item_table)
